# baseline (speedup 1.0000x reference)
import math

import jax
import jax.numpy as jnp
import numpy as np
from jax import lax
from jax.experimental import pallas as pl
from jax.experimental.pallas import tpu as pltpu
from jax.experimental.pallas import tpu_sc as plsc

D = 64
B = 4096
S = 200
NC, NS = 2, 16
NW = NC * NS
ROWS_PER_W = B // NW
SCALE = float(D) ** 0.5


def _pe_table():
    position = np.arange(0, S, dtype=np.float32)[:, None]
    half = D // 2
    div_term = np.exp(
        np.arange(0, half, dtype=np.float32) * -(math.log(10000.0) / (half - 1))
    )
    pe = np.concatenate([np.sin(position * div_term), np.cos(position * div_term)], axis=1)
    return jnp.asarray(pe, dtype=jnp.float32)


_PE = _pe_table()


def _emb_body(x_ref, table_ref, pe_ref, out_ref, idx_v, rows_v, pe_v, sem):
    wid = lax.axis_index("s") * NC + lax.axis_index("c")
    pltpu.sync_copy(pe_ref, pe_v)

    def chunk(i, carry):
        base = (wid * ROWS_PER_W + i) * S
        pltpu.sync_copy(x_ref.at[pl.ds(base, S)], idx_v)
        pltpu.async_copy(table_ref.at[idx_v], rows_v, sem).wait()

        def row(r, c2):
            for k in range(D // 16):
                sl = pl.ds(k * 16, 16)
                rows_v[r, sl] = rows_v[r, sl] * SCALE + pe_v[r, sl]
            return c2

        lax.fori_loop(0, S, row, 0)
        pltpu.sync_copy(rows_v, out_ref.at[pl.ds(base, S)])
        return carry

    lax.fori_loop(0, ROWS_PER_W, chunk, 0)


@jax.jit
def _emb(x_flat, table, pe):
    mesh = plsc.VectorSubcoreMesh(core_axis_name="c", subcore_axis_name="s")
    k = pl.kernel(
        _emb_body,
        out_type=jax.ShapeDtypeStruct((B * S, D), jnp.float32),
        mesh=mesh,
        scratch_types=[
            pltpu.VMEM((S,), jnp.int32),
            pltpu.VMEM((S, D), jnp.float32),
            pltpu.VMEM((S, D), jnp.float32),
            pltpu.SemaphoreType.DMA,
        ],
        compiler_params=pltpu.CompilerParams(use_tc_tiling_on_sc=False),
    )
    return k(x_flat, table, pe)


def kernel(x, table):
    out = _emb(x.reshape(-1), table, _PE)
    return out.reshape(B, S, D)


# speedup vs baseline: 1.2181x; 1.2181x over previous
import math

import jax
import jax.numpy as jnp
import numpy as np
from jax import lax
from jax.experimental import pallas as pl
from jax.experimental.pallas import tpu as pltpu
from jax.experimental.pallas import tpu_sc as plsc

D = 64
B = 4096
S = 200
NC, NS = 2, 16
NW = NC * NS
SPAN = B * S // NW
C = 2 * S
CHUNKS = SPAN // C
NJ = CHUNKS // 2
SCALE = float(D) ** 0.5


def _pe_table():
    position = np.arange(0, S, dtype=np.float32)[:, None]
    half = D // 2
    div_term = np.exp(
        np.arange(0, half, dtype=np.float32) * -(math.log(10000.0) / (half - 1))
    )
    pe = np.concatenate([np.sin(position * div_term), np.cos(position * div_term)], axis=1)
    return jnp.asarray(pe, dtype=jnp.float32)


_PE = _pe_table()


def _emb_body(x_ref, table_ref, pe_ref, out_ref,
              idx_all, rows0, rows1, pe_v,
              gsem0, gsem1, ssem0, ssem1):
    wid = lax.axis_index("s") * NC + lax.axis_index("c")
    wbase = wid * SPAN

    pltpu.sync_copy(x_ref.at[pl.ds(wbase, SPAN)], idx_all)
    pltpu.sync_copy(pe_ref, pe_v)

    def gather(c, rows, gsem):
        pltpu.async_copy(table_ref.at[idx_all.at[pl.ds(c * C, C)]], rows, gsem)

    def wait_gather(rows, gsem):
        pltpu.make_async_copy(table_ref.at[idx_all.at[pl.ds(0, C)]], rows, gsem).wait()

    def store(c, rows, ssem):
        pltpu.async_copy(rows, out_ref.at[pl.ds(wbase + c * C, C)], ssem)

    def wait_store(rows, ssem):
        pltpu.make_async_copy(rows, out_ref.at[pl.ds(wbase, C)], ssem).wait()

    def compute(rows):
        @plsc.parallel_loop(0, S, unroll=4)
        def _(r):
            for k in range(D // 16):
                sl = pl.ds(k * 16, 16)
                p = pe_v[r, sl]
                rows[r, sl] = rows[r, sl] * SCALE + p
                rows[S + r, sl] = rows[S + r, sl] * SCALE + p

    gather(0, rows0, gsem0)

    def body(j, carry):
        c0 = 2 * j

        @pl.when(j > 0)
        def _():
            wait_store(rows1, ssem1)
        gather(c0 + 1, rows1, gsem1)

        wait_gather(rows0, gsem0)
        compute(rows0)
        store(c0, rows0, ssem0)

        @pl.when(j + 1 < NJ)
        def _():
            wait_store(rows0, ssem0)
            gather(c0 + 2, rows0, gsem0)

        wait_gather(rows1, gsem1)
        compute(rows1)
        store(c0 + 1, rows1, ssem1)
        return carry

    lax.fori_loop(0, NJ, body, 0)

    wait_store(rows0, ssem0)
    wait_store(rows1, ssem1)


@jax.jit
def _emb(x_flat, table, pe):
    mesh = plsc.VectorSubcoreMesh(core_axis_name="c", subcore_axis_name="s")
    k = pl.kernel(
        _emb_body,
        out_type=jax.ShapeDtypeStruct((B * S, D), jnp.float32),
        mesh=mesh,
        scratch_types=[
            pltpu.VMEM((SPAN,), jnp.int32),
            pltpu.VMEM((C, D), jnp.float32),
            pltpu.VMEM((C, D), jnp.float32),
            pltpu.VMEM((S, D), jnp.float32),
            pltpu.SemaphoreType.DMA,
            pltpu.SemaphoreType.DMA,
            pltpu.SemaphoreType.DMA,
            pltpu.SemaphoreType.DMA,
        ],
        compiler_params=pltpu.CompilerParams(use_tc_tiling_on_sc=False),
    )
    return k(x_flat, table, pe)


def kernel(x, table):
    out = _emb(x.reshape(-1), table, _PE)
    return out.reshape(B, S, D)


# speedup vs baseline: 1.4934x; 1.2260x over previous
import math

import jax
import jax.numpy as jnp
import numpy as np
from jax import lax
from jax.experimental import pallas as pl
from jax.experimental.pallas import tpu as pltpu
from jax.experimental.pallas import tpu_sc as plsc

D = 64
DP = 128
B = 4096
S = 200
NC, NS = 2, 16
NW = NC * NS
SPAN = B * S // NW
C = S
CHUNKS = SPAN // C
NJ = CHUNKS // 2
SCALE = float(D) ** 0.5


def _pe_table():
    position = np.arange(0, S, dtype=np.float32)[:, None]
    half = D // 2
    div_term = np.exp(
        np.arange(0, half, dtype=np.float32) * -(math.log(10000.0) / (half - 1))
    )
    pe = np.concatenate([np.sin(position * div_term), np.cos(position * div_term)], axis=1)
    return jnp.asarray(pe, dtype=jnp.float32)


_PE = _pe_table()


def _emb_body(x_ref, table_ref, pe_ref, out_ref,
              idx_all, rows0, rows1, outbuf, pe_v,
              gsem0, gsem1, ssem):
    wid = lax.axis_index("s") * NC + lax.axis_index("c")
    wbase = wid * SPAN

    pltpu.sync_copy(x_ref.at[pl.ds(wbase, SPAN)], idx_all)
    pltpu.sync_copy(pe_ref, pe_v)

    def gather(c, rows, gsem):
        pltpu.async_copy(table_ref.at[idx_all.at[pl.ds(c * C, C)]], rows, gsem)

    def wait_gather(rows, gsem):
        pltpu.make_async_copy(
            table_ref.at[idx_all.at[pl.ds(0, C)]], rows, gsem).wait()

    def store(c):
        pltpu.async_copy(outbuf, out_ref.at[pl.ds(wbase + c * C, C)], ssem)

    def wait_store():
        pltpu.make_async_copy(outbuf, out_ref.at[pl.ds(wbase, C)], ssem).wait()

    def compute(rows):
        @plsc.parallel_loop(0, C, unroll=4)
        def _(r):
            for k in range(D // 16):
                sl = pl.ds(k * 16, 16)
                outbuf[r, sl] = rows[r, sl] * SCALE + pe_v[r, sl]

    gather(0, rows0, gsem0)

    def body(j, carry):
        c0 = 2 * j

        gather(c0 + 1, rows1, gsem1)

        wait_gather(rows0, gsem0)
        @pl.when(j > 0)
        def _():
            wait_store()
        compute(rows0)
        store(c0)

        @pl.when(j + 1 < NJ)
        def _():
            gather(c0 + 2, rows0, gsem0)

        wait_gather(rows1, gsem1)
        wait_store()
        compute(rows1)
        store(c0 + 1)
        return carry

    lax.fori_loop(0, NJ, body, 0)

    wait_store()


@jax.jit
def _emb(x_flat, table_padded, pe):
    mesh = plsc.VectorSubcoreMesh(core_axis_name="c", subcore_axis_name="s")
    k = pl.kernel(
        _emb_body,
        out_type=jax.ShapeDtypeStruct((B * S, D), jnp.float32),
        mesh=mesh,
        scratch_types=[
            pltpu.VMEM((SPAN,), jnp.int32),
            pltpu.VMEM((C, DP), jnp.float32),
            pltpu.VMEM((C, DP), jnp.float32),
            pltpu.VMEM((C, D), jnp.float32),
            pltpu.VMEM((S, D), jnp.float32),
            pltpu.SemaphoreType.DMA,
            pltpu.SemaphoreType.DMA,
            pltpu.SemaphoreType.DMA,
        ],
        compiler_params=pltpu.CompilerParams(use_tc_tiling_on_sc=True),
    )
    return k(x_flat, table_padded, pe)


def kernel(x, table):
    tab128 = jnp.pad(table, ((0, 0), (0, DP - D)))
    out = _emb(x.reshape(-1), tab128, _PE)
    return out.reshape(B, S, D)


# speedup vs baseline: 1.4955x; 1.0014x over previous
import math

import jax
import jax.numpy as jnp
import numpy as np
from jax import lax
from jax.experimental import pallas as pl
from jax.experimental.pallas import tpu as pltpu
from jax.experimental.pallas import tpu_sc as plsc

D = 64
DP = 128
B = 4096
S = 200
NC, NS = 2, 16
NW = NC * NS
SPAN = B * S // NW
C = S
CHUNKS = SPAN // C
NJ = CHUNKS // 2
SCALE = float(D) ** 0.5


def _pe_table():
    position = np.arange(0, S, dtype=np.float32)[:, None]
    half = D // 2
    div_term = np.exp(
        np.arange(0, half, dtype=np.float32) * -(math.log(10000.0) / (half - 1))
    )
    pe = np.concatenate([np.sin(position * div_term), np.cos(position * div_term)], axis=1)
    return jnp.asarray(pe, dtype=jnp.float32)


_PE = _pe_table()


def _emb_body(x_ref, table_ref, pe_ref, out_ref,
              idx0, idx1, rows0, rows1, out0, out1, pe_v,
              isem0, isem1, gsem0, gsem1, ssem0, ssem1):
    wid = lax.axis_index("s") * NC + lax.axis_index("c")
    wbase = wid * SPAN

    pltpu.sync_copy(pe_ref, pe_v)

    def idx_copy(c, idx, isem):
        pltpu.async_copy(x_ref.at[pl.ds(wbase + c * C, C)], idx, isem)

    def wait_idx(idx, isem):
        pltpu.make_async_copy(x_ref.at[pl.ds(wbase, C)], idx, isem).wait()

    def gather(idx, rows, gsem):
        pltpu.async_copy(table_ref.at[idx], rows, gsem)

    def wait_gather(idx, rows, gsem):
        pltpu.make_async_copy(table_ref.at[idx], rows, gsem).wait()

    def store(c, outb, ssem):
        pltpu.async_copy(outb, out_ref.at[pl.ds(wbase + c * C, C)], ssem)

    def wait_store(outb, ssem):
        pltpu.make_async_copy(outb, out_ref.at[pl.ds(wbase, C)], ssem).wait()

    def compute(rows, outb):
        @plsc.parallel_loop(0, C, unroll=4)
        def _(r):
            for k in range(D // 16):
                sl = pl.ds(k * 16, 16)
                outb[r, sl] = rows[r, sl] * SCALE + pe_v[r, sl]

    pltpu.sync_copy(x_ref.at[pl.ds(wbase, C)], idx0)
    gather(idx0, rows0, gsem0)
    idx_copy(1, idx1, isem1)

    def body(j, carry):
        a = 2 * j

        wait_idx(idx1, isem1)
        gather(idx1, rows1, gsem1)

        wait_gather(idx0, rows0, gsem0)
        @pl.when(j + 1 < NJ)
        def _():
            idx_copy(a + 2, idx0, isem0)
        @pl.when(j > 0)
        def _():
            wait_store(out0, ssem0)
        compute(rows0, out0)
        store(a, out0, ssem0)

        @pl.when(j + 1 < NJ)
        def _():
            wait_idx(idx0, isem0)
            gather(idx0, rows0, gsem0)

        wait_gather(idx1, rows1, gsem1)
        @pl.when(j + 1 < NJ)
        def _():
            idx_copy(a + 3, idx1, isem1)
        @pl.when(j > 0)
        def _():
            wait_store(out1, ssem1)
        compute(rows1, out1)
        store(a + 1, out1, ssem1)
        return carry

    lax.fori_loop(0, NJ, body, 0)

    wait_store(out0, ssem0)
    wait_store(out1, ssem1)


@jax.jit
def _emb(x_flat, table_padded, pe):
    mesh = plsc.VectorSubcoreMesh(core_axis_name="c", subcore_axis_name="s")
    k = pl.kernel(
        _emb_body,
        out_type=jax.ShapeDtypeStruct((B * S, D), jnp.float32),
        mesh=mesh,
        scratch_types=[
            pltpu.VMEM((C,), jnp.int32),
            pltpu.VMEM((C,), jnp.int32),
            pltpu.VMEM((C, DP), jnp.float32),
            pltpu.VMEM((C, DP), jnp.float32),
            pltpu.VMEM((C, D), jnp.float32),
            pltpu.VMEM((C, D), jnp.float32),
            pltpu.VMEM((S, D), jnp.float32),
            pltpu.SemaphoreType.DMA,
            pltpu.SemaphoreType.DMA,
            pltpu.SemaphoreType.DMA,
            pltpu.SemaphoreType.DMA,
            pltpu.SemaphoreType.DMA,
            pltpu.SemaphoreType.DMA,
        ],
        compiler_params=pltpu.CompilerParams(use_tc_tiling_on_sc=True),
    )
    return k(x_flat, table_padded, pe)


def kernel(x, table):
    tab128 = jnp.pad(table, ((0, 0), (0, DP - D)))
    out = _emb(x.reshape(-1), tab128, _PE)
    return out.reshape(B, S, D)


# speedup vs baseline: 1.9862x; 1.3281x over previous
import math

import jax
import jax.numpy as jnp
import numpy as np
from jax import lax
from jax.experimental import pallas as pl
from jax.experimental.pallas import tpu as pltpu
from jax.experimental.pallas import tpu_sc as plsc

D = 64
DP = 128
VOCAB = 1000000
B = 4096
S = 200
NC, NS = 2, 16
NW = NC * NS
SPAN = B * S // NW
C = S
CHUNKS = SPAN // C
NJ = CHUNKS // 2
SCALE = float(D) ** 0.5


def _pe_table():
    position = np.arange(0, S, dtype=np.float32)[:, None]
    half = D // 2
    div_term = np.exp(
        np.arange(0, half, dtype=np.float32) * -(math.log(10000.0) / (half - 1))
    )
    pe = np.concatenate([np.sin(position * div_term), np.cos(position * div_term)], axis=1)
    return jnp.asarray(pe, dtype=jnp.float32)


_PE = _pe_table()

TN = 8192


def _tp_body(tt_ref, out_ref):
    blk = tt_ref[...]
    eye = jax.lax.broadcasted_iota(jnp.int32, (D, D), 0) == \
        jax.lax.broadcasted_iota(jnp.int32, (D, D), 1)
    t = jax.lax.dot_general(blk, eye.astype(jnp.float32),
                            (((0,), (0,)), ((), ())),
                            preferred_element_type=jnp.float32)
    out_ref[:, 0:D] = t * SCALE
    out_ref[:, D:DP] = jnp.zeros((TN, D), jnp.float32)


@jax.jit
def _transpose_pad(table_t):
    grid = (VOCAB + TN - 1) // TN
    return pl.pallas_call(
        _tp_body,
        grid=(grid,),
        in_specs=[pl.BlockSpec((D, TN), lambda j: (0, j))],
        out_specs=pl.BlockSpec((TN, DP), lambda j: (j, 0)),
        out_shape=jax.ShapeDtypeStruct((VOCAB, DP), jnp.float32),
    )(table_t)


def _emb_body(x_ref, table_ref, pe_ref, out_ref,
              idx0, idx1, rows0, rows1, out0, out1, pe_v,
              isem0, isem1, gsem0, gsem1, ssem0, ssem1):
    wid = lax.axis_index("s") * NC + lax.axis_index("c")
    wbase = wid * SPAN

    pltpu.sync_copy(pe_ref, pe_v)

    def idx_copy(c, idx, isem):
        pltpu.async_copy(x_ref.at[pl.ds(wbase + c * C, C)], idx, isem)

    def wait_idx(idx, isem):
        pltpu.make_async_copy(x_ref.at[pl.ds(wbase, C)], idx, isem).wait()

    def gather(idx, rows, gsem):
        pltpu.async_copy(table_ref.at[idx], rows, gsem)

    def wait_gather(idx, rows, gsem):
        pltpu.make_async_copy(table_ref.at[idx], rows, gsem).wait()

    def store(c, outb, ssem):
        pltpu.async_copy(outb, out_ref.at[pl.ds(wbase + c * C, C)], ssem)

    def wait_store(outb, ssem):
        pltpu.make_async_copy(outb, out_ref.at[pl.ds(wbase, C)], ssem).wait()

    def compute(rows, outb):
        @plsc.parallel_loop(0, C, unroll=4)
        def _(r):
            for k in range(D // 16):
                sl = pl.ds(k * 16, 16)
                outb[r, sl] = rows[r, sl] + pe_v[r, sl]

    pltpu.sync_copy(x_ref.at[pl.ds(wbase, C)], idx0)
    gather(idx0, rows0, gsem0)
    idx_copy(1, idx1, isem1)

    def body(j, carry):
        a = 2 * j

        wait_idx(idx1, isem1)
        gather(idx1, rows1, gsem1)

        wait_gather(idx0, rows0, gsem0)
        @pl.when(j + 1 < NJ)
        def _():
            idx_copy(a + 2, idx0, isem0)
        @pl.when(j > 0)
        def _():
            wait_store(out0, ssem0)
        compute(rows0, out0)
        store(a, out0, ssem0)

        @pl.when(j + 1 < NJ)
        def _():
            wait_idx(idx0, isem0)
            gather(idx0, rows0, gsem0)

        wait_gather(idx1, rows1, gsem1)
        @pl.when(j + 1 < NJ)
        def _():
            idx_copy(a + 3, idx1, isem1)
        @pl.when(j > 0)
        def _():
            wait_store(out1, ssem1)
        compute(rows1, out1)
        store(a + 1, out1, ssem1)
        return carry

    lax.fori_loop(0, NJ, body, 0)

    wait_store(out0, ssem0)
    wait_store(out1, ssem1)


@jax.jit
def _emb(x_flat, table_padded, pe):
    mesh = plsc.VectorSubcoreMesh(core_axis_name="c", subcore_axis_name="s")
    k = pl.kernel(
        _emb_body,
        out_type=jax.ShapeDtypeStruct((B * S, D), jnp.float32),
        mesh=mesh,
        scratch_types=[
            pltpu.VMEM((C,), jnp.int32),
            pltpu.VMEM((C,), jnp.int32),
            pltpu.VMEM((C, DP), jnp.float32),
            pltpu.VMEM((C, DP), jnp.float32),
            pltpu.VMEM((C, D), jnp.float32),
            pltpu.VMEM((C, D), jnp.float32),
            pltpu.VMEM((S, D), jnp.float32),
            pltpu.SemaphoreType.DMA,
            pltpu.SemaphoreType.DMA,
            pltpu.SemaphoreType.DMA,
            pltpu.SemaphoreType.DMA,
            pltpu.SemaphoreType.DMA,
            pltpu.SemaphoreType.DMA,
        ],
        compiler_params=pltpu.CompilerParams(use_tc_tiling_on_sc=True),
    )
    return k(x_flat, table_padded, pe)


def kernel(x, table):
    tab128 = _transpose_pad(table.T)
    out = _emb(x.reshape(-1), tab128, _PE)
    return out.reshape(B, S, D)


# speedup vs baseline: 2.0046x; 1.0093x over previous
import math

import jax
import jax.numpy as jnp
import numpy as np
from jax import lax
from jax.experimental import pallas as pl
from jax.experimental.pallas import tpu as pltpu
from jax.experimental.pallas import tpu_sc as plsc

D = 64
DP = 128
VOCAB = 1000000
B = 4096
S = 200
NC, NS = 2, 16
NW = NC * NS
SPAN = B * S // NW
C = S
CHUNKS = SPAN // C
NJ = CHUNKS // 2
SCALE = float(D) ** 0.5


def _pe_table():
    position = np.arange(0, S, dtype=np.float32)[:, None]
    half = D // 2
    div_term = np.exp(
        np.arange(0, half, dtype=np.float32) * -(math.log(10000.0) / (half - 1))
    )
    pe = np.concatenate([np.sin(position * div_term), np.cos(position * div_term)], axis=1)
    return jnp.asarray(pe, dtype=jnp.float32)


_PE = _pe_table()

TN = 8192


def _tp_body(tt_ref, out_ref):
    t = jnp.transpose(tt_ref[...], (1, 0))
    out_ref[:, 0:D] = t * SCALE
    out_ref[:, D:DP] = jnp.zeros((TN, D), jnp.float32)


@jax.jit
def _transpose_pad(table_t):
    grid = (VOCAB + TN - 1) // TN
    return pl.pallas_call(
        _tp_body,
        grid=(grid,),
        in_specs=[pl.BlockSpec((D, TN), lambda j: (0, j))],
        out_specs=pl.BlockSpec((TN, DP), lambda j: (j, 0)),
        out_shape=jax.ShapeDtypeStruct((VOCAB, DP), jnp.float32),
    )(table_t)


def _emb_body(x_ref, table_ref, pe_ref, out_ref,
              idx0, idx1, rows0, rows1, out0, out1, pe_v,
              isem0, isem1, gsem0, gsem1, ssem0, ssem1):
    wid = lax.axis_index("s") * NC + lax.axis_index("c")
    wbase = wid * SPAN

    pltpu.sync_copy(pe_ref, pe_v)

    def idx_copy(c, idx, isem):
        pltpu.async_copy(x_ref.at[pl.ds(wbase + c * C, C)], idx, isem)

    def wait_idx(idx, isem):
        pltpu.make_async_copy(x_ref.at[pl.ds(wbase, C)], idx, isem).wait()

    def gather(idx, rows, gsem):
        pltpu.async_copy(table_ref.at[idx], rows, gsem)

    def wait_gather(idx, rows, gsem):
        pltpu.make_async_copy(table_ref.at[idx], rows, gsem).wait()

    def store(c, outb, ssem):
        pltpu.async_copy(outb, out_ref.at[pl.ds(wbase + c * C, C)], ssem)

    def wait_store(outb, ssem):
        pltpu.make_async_copy(outb, out_ref.at[pl.ds(wbase, C)], ssem).wait()

    def compute(rows, outb):
        @plsc.parallel_loop(0, C, unroll=4)
        def _(r):
            for k in range(D // 16):
                sl = pl.ds(k * 16, 16)
                outb[r, sl] = rows[r, sl] + pe_v[r, sl]

    pltpu.sync_copy(x_ref.at[pl.ds(wbase, C)], idx0)
    gather(idx0, rows0, gsem0)
    idx_copy(1, idx1, isem1)

    def body(j, carry):
        a = 2 * j

        wait_idx(idx1, isem1)
        gather(idx1, rows1, gsem1)

        wait_gather(idx0, rows0, gsem0)
        @pl.when(j + 1 < NJ)
        def _():
            idx_copy(a + 2, idx0, isem0)
        @pl.when(j > 0)
        def _():
            wait_store(out0, ssem0)
        compute(rows0, out0)
        store(a, out0, ssem0)

        @pl.when(j + 1 < NJ)
        def _():
            wait_idx(idx0, isem0)
            gather(idx0, rows0, gsem0)

        wait_gather(idx1, rows1, gsem1)
        @pl.when(j + 1 < NJ)
        def _():
            idx_copy(a + 3, idx1, isem1)
        @pl.when(j > 0)
        def _():
            wait_store(out1, ssem1)
        compute(rows1, out1)
        store(a + 1, out1, ssem1)
        return carry

    lax.fori_loop(0, NJ, body, 0)

    wait_store(out0, ssem0)
    wait_store(out1, ssem1)


@jax.jit
def _emb(x_flat, table_padded, pe):
    mesh = plsc.VectorSubcoreMesh(core_axis_name="c", subcore_axis_name="s")
    k = pl.kernel(
        _emb_body,
        out_type=jax.ShapeDtypeStruct((B * S, D), jnp.float32),
        mesh=mesh,
        scratch_types=[
            pltpu.VMEM((C,), jnp.int32),
            pltpu.VMEM((C,), jnp.int32),
            pltpu.VMEM((C, DP), jnp.float32),
            pltpu.VMEM((C, DP), jnp.float32),
            pltpu.VMEM((C, D), jnp.float32),
            pltpu.VMEM((C, D), jnp.float32),
            pltpu.VMEM((S, D), jnp.float32),
            pltpu.SemaphoreType.DMA,
            pltpu.SemaphoreType.DMA,
            pltpu.SemaphoreType.DMA,
            pltpu.SemaphoreType.DMA,
            pltpu.SemaphoreType.DMA,
            pltpu.SemaphoreType.DMA,
        ],
        compiler_params=pltpu.CompilerParams(use_tc_tiling_on_sc=True),
    )
    return k(x_flat, table_padded, pe)


def kernel(x, table):
    tab128 = _transpose_pad(table.T)
    out = _emb(x.reshape(-1), tab128, _PE)
    return out.reshape(B, S, D)


# speedup vs baseline: 2.0531x; 1.0242x over previous
import math

import jax
import jax.numpy as jnp
import numpy as np
from jax import lax
from jax.experimental import pallas as pl
from jax.experimental.pallas import tpu as pltpu
from jax.experimental.pallas import tpu_sc as plsc

D = 64
DP = 128
VOCAB = 1000000
B = 4096
S = 200
NC, NS = 2, 16
NW = NC * NS
SPAN = B * S // NW
C = S
CHUNKS = SPAN // C
NJ = CHUNKS // 2
SCALE = float(D) ** 0.5


def _pe_table():
    position = np.arange(0, S, dtype=np.float32)[:, None]
    half = D // 2
    div_term = np.exp(
        np.arange(0, half, dtype=np.float32) * -(math.log(10000.0) / (half - 1))
    )
    pe = np.concatenate([np.sin(position * div_term), np.cos(position * div_term)], axis=1)
    return jnp.asarray(pe, dtype=jnp.float32)


_PE = _pe_table()

TN = 16384


def _tp_body(tt_ref, out_ref):
    t = jnp.transpose(tt_ref[...], (1, 0))
    out_ref[:, 0:D] = t * SCALE
    out_ref[:, D:DP] = jnp.zeros((TN, D), jnp.float32)


@jax.jit
def _transpose_pad(table_t):
    grid = (VOCAB + TN - 1) // TN
    return pl.pallas_call(
        _tp_body,
        grid=(grid,),
        in_specs=[pl.BlockSpec((D, TN), lambda j: (0, j))],
        out_specs=pl.BlockSpec((TN, DP), lambda j: (j, 0)),
        out_shape=jax.ShapeDtypeStruct((VOCAB, DP), jnp.float32),
    )(table_t)


def _emb_body(x_ref, table_ref, pe_ref, out_ref,
              idx0, idx1, rows0, rows1, out0, out1, pe_v,
              isem0, isem1, gsem0, gsem1, ssem0, ssem1):
    wid = lax.axis_index("s") * NC + lax.axis_index("c")
    wbase = wid * SPAN

    pltpu.sync_copy(pe_ref, pe_v)

    def idx_copy(c, idx, isem):
        pltpu.async_copy(x_ref.at[pl.ds(wbase + c * C, C)], idx, isem)

    def wait_idx(idx, isem):
        pltpu.make_async_copy(x_ref.at[pl.ds(wbase, C)], idx, isem).wait()

    def gather(idx, rows, gsem):
        pltpu.async_copy(table_ref.at[idx], rows, gsem)

    def wait_gather(idx, rows, gsem):
        pltpu.make_async_copy(table_ref.at[idx], rows, gsem).wait()

    def store(c, outb, ssem):
        pltpu.async_copy(outb, out_ref.at[pl.ds(wbase + c * C, C)], ssem)

    def wait_store(outb, ssem):
        pltpu.make_async_copy(outb, out_ref.at[pl.ds(wbase, C)], ssem).wait()

    def compute(rows, outb):
        @plsc.parallel_loop(0, C, unroll=4)
        def _(r):
            for k in range(D // 16):
                sl = pl.ds(k * 16, 16)
                outb[r, sl] = rows[r, sl] + pe_v[r, sl]

    pltpu.sync_copy(x_ref.at[pl.ds(wbase, C)], idx0)
    gather(idx0, rows0, gsem0)
    idx_copy(1, idx1, isem1)

    def body(j, carry):
        a = 2 * j

        wait_idx(idx1, isem1)
        gather(idx1, rows1, gsem1)

        wait_gather(idx0, rows0, gsem0)
        @pl.when(j + 1 < NJ)
        def _():
            idx_copy(a + 2, idx0, isem0)
        @pl.when(j > 0)
        def _():
            wait_store(out0, ssem0)
        compute(rows0, out0)
        store(a, out0, ssem0)

        @pl.when(j + 1 < NJ)
        def _():
            wait_idx(idx0, isem0)
            gather(idx0, rows0, gsem0)

        wait_gather(idx1, rows1, gsem1)
        @pl.when(j + 1 < NJ)
        def _():
            idx_copy(a + 3, idx1, isem1)
        @pl.when(j > 0)
        def _():
            wait_store(out1, ssem1)
        compute(rows1, out1)
        store(a + 1, out1, ssem1)
        return carry

    lax.fori_loop(0, NJ, body, 0)

    wait_store(out0, ssem0)
    wait_store(out1, ssem1)


@jax.jit
def _emb(x_flat, table_padded, pe):
    mesh = plsc.VectorSubcoreMesh(core_axis_name="c", subcore_axis_name="s")
    k = pl.kernel(
        _emb_body,
        out_type=jax.ShapeDtypeStruct((B * S, D), jnp.float32),
        mesh=mesh,
        scratch_types=[
            pltpu.VMEM((C,), jnp.int32),
            pltpu.VMEM((C,), jnp.int32),
            pltpu.VMEM((C, DP), jnp.float32),
            pltpu.VMEM((C, DP), jnp.float32),
            pltpu.VMEM((C, D), jnp.float32),
            pltpu.VMEM((C, D), jnp.float32),
            pltpu.VMEM((S, D), jnp.float32),
            pltpu.SemaphoreType.DMA,
            pltpu.SemaphoreType.DMA,
            pltpu.SemaphoreType.DMA,
            pltpu.SemaphoreType.DMA,
            pltpu.SemaphoreType.DMA,
            pltpu.SemaphoreType.DMA,
        ],
        compiler_params=pltpu.CompilerParams(use_tc_tiling_on_sc=True),
    )
    return k(x_flat, table_padded, pe)


def kernel(x, table):
    tab128 = _transpose_pad(table.T)
    out = _emb(x.reshape(-1), tab128, _PE)
    return out.reshape(B, S, D)


# speedup vs baseline: 2.0734x; 1.0098x over previous
import math

import jax
import jax.numpy as jnp
import numpy as np
from jax import lax
from jax.experimental import pallas as pl
from jax.experimental.pallas import tpu as pltpu
from jax.experimental.pallas import tpu_sc as plsc

D = 64
DP = 128
VOCAB = 1000000
B = 4096
S = 200
NC, NS = 2, 16
NW = NC * NS
SPAN = B * S // NW
C = S
CHUNKS = SPAN // C
NJ = CHUNKS // 2
SCALE = float(D) ** 0.5


def _pe_table():
    position = np.arange(0, S, dtype=np.float32)[:, None]
    half = D // 2
    div_term = np.exp(
        np.arange(0, half, dtype=np.float32) * -(math.log(10000.0) / (half - 1))
    )
    pe = np.concatenate([np.sin(position * div_term), np.cos(position * div_term)], axis=1)
    return jnp.asarray(pe, dtype=jnp.float32)


_PE = _pe_table()

TN = 32768


def _tp_body(tt_ref, out_ref):
    t = jnp.transpose(tt_ref[...], (1, 0))
    out_ref[:, 0:D] = t * SCALE
    out_ref[:, D:DP] = jnp.zeros((TN, D), jnp.float32)


@jax.jit
def _transpose_pad(table_t):
    grid = (VOCAB + TN - 1) // TN
    return pl.pallas_call(
        _tp_body,
        grid=(grid,),
        in_specs=[pl.BlockSpec((D, TN), lambda j: (0, j))],
        out_specs=pl.BlockSpec((TN, DP), lambda j: (j, 0)),
        out_shape=jax.ShapeDtypeStruct((VOCAB, DP), jnp.float32),
    )(table_t)


def _emb_body(x_ref, table_ref, pe_ref, out_ref,
              idx0, idx1, rows0, rows1, out0, out1, pe_v,
              isem0, isem1, gsem0, gsem1, ssem0, ssem1):
    wid = lax.axis_index("s") * NC + lax.axis_index("c")
    wbase = wid * SPAN

    pltpu.sync_copy(pe_ref, pe_v)

    def idx_copy(c, idx, isem):
        pltpu.async_copy(x_ref.at[pl.ds(wbase + c * C, C)], idx, isem)

    def wait_idx(idx, isem):
        pltpu.make_async_copy(x_ref.at[pl.ds(wbase, C)], idx, isem).wait()

    def gather(idx, rows, gsem):
        pltpu.async_copy(table_ref.at[idx], rows, gsem)

    def wait_gather(idx, rows, gsem):
        pltpu.make_async_copy(table_ref.at[idx], rows, gsem).wait()

    def store(c, outb, ssem):
        pltpu.async_copy(outb, out_ref.at[pl.ds(wbase + c * C, C)], ssem)

    def wait_store(outb, ssem):
        pltpu.make_async_copy(outb, out_ref.at[pl.ds(wbase, C)], ssem).wait()

    def compute(rows, outb):
        @plsc.parallel_loop(0, C, unroll=4)
        def _(r):
            for k in range(D // 16):
                sl = pl.ds(k * 16, 16)
                outb[r, sl] = rows[r, sl] + pe_v[r, sl]

    pltpu.sync_copy(x_ref.at[pl.ds(wbase, C)], idx0)
    gather(idx0, rows0, gsem0)
    idx_copy(1, idx1, isem1)

    def body(j, carry):
        a = 2 * j

        wait_idx(idx1, isem1)
        gather(idx1, rows1, gsem1)

        wait_gather(idx0, rows0, gsem0)
        @pl.when(j + 1 < NJ)
        def _():
            idx_copy(a + 2, idx0, isem0)
        @pl.when(j > 0)
        def _():
            wait_store(out0, ssem0)
        compute(rows0, out0)
        store(a, out0, ssem0)

        @pl.when(j + 1 < NJ)
        def _():
            wait_idx(idx0, isem0)
            gather(idx0, rows0, gsem0)

        wait_gather(idx1, rows1, gsem1)
        @pl.when(j + 1 < NJ)
        def _():
            idx_copy(a + 3, idx1, isem1)
        @pl.when(j > 0)
        def _():
            wait_store(out1, ssem1)
        compute(rows1, out1)
        store(a + 1, out1, ssem1)
        return carry

    lax.fori_loop(0, NJ, body, 0)

    wait_store(out0, ssem0)
    wait_store(out1, ssem1)


@jax.jit
def _emb(x_flat, table_padded, pe):
    mesh = plsc.VectorSubcoreMesh(core_axis_name="c", subcore_axis_name="s")
    k = pl.kernel(
        _emb_body,
        out_type=jax.ShapeDtypeStruct((B * S, D), jnp.float32),
        mesh=mesh,
        scratch_types=[
            pltpu.VMEM((C,), jnp.int32),
            pltpu.VMEM((C,), jnp.int32),
            pltpu.VMEM((C, DP), jnp.float32),
            pltpu.VMEM((C, DP), jnp.float32),
            pltpu.VMEM((C, D), jnp.float32),
            pltpu.VMEM((C, D), jnp.float32),
            pltpu.VMEM((S, D), jnp.float32),
            pltpu.SemaphoreType.DMA,
            pltpu.SemaphoreType.DMA,
            pltpu.SemaphoreType.DMA,
            pltpu.SemaphoreType.DMA,
            pltpu.SemaphoreType.DMA,
            pltpu.SemaphoreType.DMA,
        ],
        compiler_params=pltpu.CompilerParams(use_tc_tiling_on_sc=True),
    )
    return k(x_flat, table_padded, pe)


def kernel(x, table):
    tab128 = _transpose_pad(table.T)
    out = _emb(x.reshape(-1), tab128, _PE)
    return out.reshape(B, S, D)
